# baseline (speedup 1.0000x reference)
import functools

import jax
import jax.numpy as jnp
from jax import lax
from jax.experimental import pallas as pl
from jax.experimental.pallas import tpu as pltpu
from jax.experimental.pallas import tpu_sc as plsc

_SCALE = 0.0625
_AH = 7
_AW = 7
_NS = _AH * _AW
_B, _C, _H, _W = 2, 256, 64, 64
_NPIX = _B * _H * _W
_NROI = 2000
_ROWS_PAD = 112
_OIDX_PAD = 104
_OUTROWS = _NS * (_NROI // 8) * 2 * 8


def _relayout_body(fin, fout):
    fout[...] = fin[0].T


_relayout_tc = pl.pallas_call(
    _relayout_body,
    grid=(_B, 2),
    in_specs=[pl.BlockSpec((1, 128, _H * _W), lambda b, q: (b, q, 0))],
    out_specs=pl.BlockSpec((_H * _W, 128), lambda b, q: (q * _B + b, 0)),
    out_shape=jax.ShapeDtypeStruct((2 * _NPIX, 128), jnp.float32),
)


def _roi_align_body(ftab, roisp, out, roibuf, idx0, idx1, idx2, idx3, oidx,
                    wbuf, gbuf0, gbuf1, gbuf2, gbuf3, outbuf, sem):
    cid = lax.axis_index("c")
    sid = lax.axis_index("s")
    wid = sid * 2 + cid
    start = wid * 62 + jnp.minimum(wid, 16)
    pltpu.sync_copy(roisp.at[pl.ds(start * 16, 64 * 16)], roibuf)

    iota = lax.iota(jnp.int32, 16)
    idxbufs = (idx0, idx1, idx2, idx3)
    gbufs = (gbuf0, gbuf1, gbuf2, gbuf3)
    for q in range(4):
        idxbufs[q][pl.ds(_ROWS_PAD - 16, 16)] = jnp.zeros((16,), jnp.int32)

    def _splat_load(ref, i):
        return plsc.load_gather(ref, [jnp.broadcast_to(i, (16,))])

    def roi_body(j, carry):
        n = jnp.minimum(start + j, _NROI - 1)
        local = (n - start) * 16
        bv = _splat_load(roibuf, local).astype(jnp.int32)
        x1 = _splat_load(roibuf, local + 1) * _SCALE
        y1 = _splat_load(roibuf, local + 2) * _SCALE
        x2 = _splat_load(roibuf, local + 3) * _SCALE
        y2 = _splat_load(roibuf, local + 4) * _SCALE
        binh = jnp.maximum(y2 - y1 + 1.0, 0.0) * (1.0 / (_AH - 1))
        binw = jnp.maximum(x2 - x1 + 1.0, 0.0) * (1.0 / (_AW - 1))

        for r in range(4):
            s = iota + 16 * r
            ph = (s // _AW).astype(jnp.float32)
            pw = (s % _AW).astype(jnp.float32)
            hs = y1 + ph * binh
            ws = x1 + pw * binw
            valid = (hs >= 0.0) & (hs < float(_H)) & (ws >= 0.0) & (ws < float(_W))
            hst = jnp.clip(hs.astype(jnp.int32), 0, _H - 2)
            wst = jnp.clip(ws.astype(jnp.int32), 0, _W - 2)
            hr = hs - hst.astype(jnp.float32)
            wr = ws - wst.astype(jnp.float32)
            vf = jnp.where(valid, 1.0, 0.0)
            omh = (1.0 - hr) * vf
            hrv = hr * vf
            wbuf[pl.ds(16 * r, 16)] = omh * (1.0 - wr)
            wbuf[pl.ds(64 + 16 * r, 16)] = omh * wr
            wbuf[pl.ds(128 + 16 * r, 16)] = hrv * (1.0 - wr)
            wbuf[pl.ds(192 + 16 * r, 16)] = hrv * wr
            p = bv * (_H * _W) + hst * _W + wst
            m = s < _NS
            plsc.store_scatter(idx0, [s], p, mask=m)
            plsc.store_scatter(idx0, [s + _NS], p + _W, mask=m)
            plsc.store_scatter(idx1, [s], p + 1, mask=m)
            plsc.store_scatter(idx1, [s + _NS], p + _W + 1, mask=m)
            plsc.store_scatter(idx2, [s], p + _NPIX, mask=m)
            plsc.store_scatter(idx2, [s + _NS], p + _NPIX + _W, mask=m)
            plsc.store_scatter(idx3, [s], p + _NPIX + 1, mask=m)
            plsc.store_scatter(idx3, [s + _NS], p + _NPIX + _W + 1, mask=m)

        base_n = (n // 8) * 16 + (n % 8)
        for r in range(7):
            pos = iota + 16 * r
            sv = pos // 2
            tcv = pos % 2
            val = sv * (2 * _NROI) + tcv * 8 + base_n
            val = jnp.where(pos < 2 * _NS, val, base_n)
            plsc.store_scatter(oidx, [pos], val, mask=pos < _OIDX_PAD)

        cps = []
        for q in range(4):
            for c in range(7):
                iv = idxbufs[q][pl.ds(c * 16, 16)]
                cps.append(pltpu.async_copy(
                    ftab.at[iv], gbufs[q].at[pl.ds(c * 16, 16), :], sem))
        for cp in cps:
            cp.wait()

        def s_body(s, c2):
            w0 = _splat_load(wbuf, s)
            w1 = _splat_load(wbuf, s + 64)
            w2 = _splat_load(wbuf, s + 128)
            w3 = _splat_load(wbuf, s + 192)
            orow = s * 2
            for k in range(_C // 16):
                ga = gbufs[0] if k < 8 else gbufs[2]
                gb = gbufs[1] if k < 8 else gbufs[3]
                off = (k % 8) * 16
                ul = ga[s, pl.ds(off, 16)]
                ur = gb[s, pl.ds(off, 16)]
                ll = ga[s + _NS, pl.ds(off, 16)]
                lr = gb[s + _NS, pl.ds(off, 16)]
                acc = ul * w0 + ur * w1 + ll * w2 + lr * w3
                outbuf[orow + (k // 8), pl.ds(off, 16)] = acc
            return c2

        lax.fori_loop(0, _NS, s_body, 0)
        for r in range(6):
            for h in range(8):
                outbuf[2 * _NS + r, pl.ds(h * 16, 16)] = (
                    outbuf[0, pl.ds(h * 16, 16)])
        pltpu.async_copy(outbuf, out.at[oidx], sem).wait()
        return carry

    lax.fori_loop(0, 63, roi_body, 0)


_roi_align_sc = functools.partial(
    pl.kernel,
    out_type=jax.ShapeDtypeStruct((_OUTROWS, 128), jnp.float32),
    mesh=plsc.VectorSubcoreMesh(core_axis_name="c", subcore_axis_name="s"),
    compiler_params=pltpu.CompilerParams(needs_layout_passes=False),
    scratch_types=[
        pltpu.VMEM((64 * 16,), jnp.float32),
        pltpu.VMEM((_ROWS_PAD,), jnp.int32),
        pltpu.VMEM((_ROWS_PAD,), jnp.int32),
        pltpu.VMEM((_ROWS_PAD,), jnp.int32),
        pltpu.VMEM((_ROWS_PAD,), jnp.int32),
        pltpu.VMEM((_OIDX_PAD,), jnp.int32),
        pltpu.VMEM((4 * 64,), jnp.float32),
        pltpu.VMEM((_ROWS_PAD, 128), jnp.float32),
        pltpu.VMEM((_ROWS_PAD, 128), jnp.float32),
        pltpu.VMEM((_ROWS_PAD, 128), jnp.float32),
        pltpu.VMEM((_ROWS_PAD, 128), jnp.float32),
        pltpu.VMEM((_OIDX_PAD, 128), jnp.float32),
        pltpu.SemaphoreType.DMA,
    ],
)(_roi_align_body)


def kernel(features, rois):
    B, C, H, W = features.shape
    n = rois.shape[0]
    ftab = _relayout_tc(features.reshape(B, C, H * W))
    roisp = jnp.zeros((2048, 16), jnp.float32).at[:n, :5].set(rois).reshape(-1)
    out = _roi_align_sc(ftab, roisp)
    out = out.reshape(_NS, _NROI // 8, 2, 8, 128)
    out = jnp.transpose(out, (1, 3, 2, 4, 0))
    return out.reshape(n, C, _AH, _AW)


# speedup vs baseline: 5.5995x; 5.5995x over previous
import functools

import jax
import jax.numpy as jnp
from jax import lax
from jax.experimental import pallas as pl
from jax.experimental.pallas import tpu as pltpu
from jax.experimental.pallas import tpu_sc as plsc

_SCALE = 0.0625
_AH = 7
_AW = 7
_NS = _AH * _AW
_B, _C, _H, _W = 2, 256, 64, 64
_NPIX = _B * _H * _W
_NROI = 2000
_ROWS_PAD = 104
_OIDX_PAD = 104
_OUTROWS = _NS * (_NROI // 8) * 2 * 8


def _relayout_body(fin, fout):
    t = fin[0].T
    tshift = jnp.concatenate([t[_W:], t[:_W]], axis=0)
    fout[...] = jnp.concatenate([t, tshift], axis=1)


_relayout_tc = pl.pallas_call(
    _relayout_body,
    grid=(_B,),
    in_specs=[pl.BlockSpec((1, _C, _H * _W), lambda b: (b, 0, 0))],
    out_specs=pl.BlockSpec((_H * _W, 2 * _C), lambda b: (b, 0)),
    out_shape=jax.ShapeDtypeStruct((_NPIX, 2 * _C), jnp.float32),
)


def _roi_align_body(ftab, roisp, out, roibuf, idxbuf, oidx,
                    wbuf, gbuf, outbuf, sem):
    cid = lax.axis_index("c")
    sid = lax.axis_index("s")
    wid = sid * 2 + cid
    start = wid * 62 + jnp.minimum(wid, 16)
    pltpu.sync_copy(roisp.at[pl.ds(start * 16, 64 * 16)], roibuf)

    iota = lax.iota(jnp.int32, 16)
    idxbuf[pl.ds(_ROWS_PAD - 16, 16)] = jnp.zeros((16,), jnp.int32)

    def _splat_load(ref, i):
        return plsc.load_gather(ref, [jnp.broadcast_to(i, (16,))])

    def roi_body(j, carry):
        n = jnp.minimum(start + j, _NROI - 1)
        local = (n - start) * 16
        bv = _splat_load(roibuf, local).astype(jnp.int32)
        x1 = _splat_load(roibuf, local + 1) * _SCALE
        y1 = _splat_load(roibuf, local + 2) * _SCALE
        x2 = _splat_load(roibuf, local + 3) * _SCALE
        y2 = _splat_load(roibuf, local + 4) * _SCALE
        binh = jnp.maximum(y2 - y1 + 1.0, 0.0) * (1.0 / (_AH - 1))
        binw = jnp.maximum(x2 - x1 + 1.0, 0.0) * (1.0 / (_AW - 1))

        for r in range(4):
            s = iota + 16 * r
            ph = (s // _AW).astype(jnp.float32)
            pw = (s % _AW).astype(jnp.float32)
            hs = y1 + ph * binh
            ws = x1 + pw * binw
            valid = (hs >= 0.0) & (hs < float(_H)) & (ws >= 0.0) & (ws < float(_W))
            hst = jnp.clip(hs.astype(jnp.int32), 0, _H - 2)
            wst = jnp.clip(ws.astype(jnp.int32), 0, _W - 2)
            hr = hs - hst.astype(jnp.float32)
            wr = ws - wst.astype(jnp.float32)
            vf = jnp.where(valid, 1.0, 0.0)
            omh = (1.0 - hr) * vf
            hrv = hr * vf
            wbuf[pl.ds(16 * r, 16)] = omh * (1.0 - wr)
            wbuf[pl.ds(64 + 16 * r, 16)] = omh * wr
            wbuf[pl.ds(128 + 16 * r, 16)] = hrv * (1.0 - wr)
            wbuf[pl.ds(192 + 16 * r, 16)] = hrv * wr
            p = bv * (_H * _W) + hst * _W + wst
            m = s < _NS
            plsc.store_scatter(idxbuf, [s], p, mask=m)
            plsc.store_scatter(idxbuf, [s + _NS], p + 1, mask=m)

        base_n = (n // 8) * 16 + (n % 8)
        for r in range(7):
            pos = iota + 16 * r
            sv = pos // 2
            tcv = pos % 2
            val = sv * (2 * _NROI) + tcv * 8 + base_n
            val = jnp.where(pos < 2 * _NS, val, base_n)
            plsc.store_scatter(oidx, [pos], val, mask=pos < _OIDX_PAD)

        pltpu.async_copy(ftab.at[idxbuf], gbuf, sem).wait()

        def s_body(s, c2):
            w0 = _splat_load(wbuf, s)
            w1 = _splat_load(wbuf, s + 64)
            w2 = _splat_load(wbuf, s + 128)
            w3 = _splat_load(wbuf, s + 192)
            orow = s * 2
            for k in range(_C // 16):
                off = 16 * k
                ul = gbuf[s, pl.ds(off, 16)]
                ur = gbuf[s + _NS, pl.ds(off, 16)]
                ll = gbuf[s, pl.ds(_C + off, 16)]
                lr = gbuf[s + _NS, pl.ds(_C + off, 16)]
                acc = ul * w0 + ur * w1 + ll * w2 + lr * w3
                outbuf[orow + (k // 8), pl.ds((k % 8) * 16, 16)] = acc
            return c2

        lax.fori_loop(0, _NS, s_body, 0)
        for r in range(6):
            for h in range(8):
                outbuf[2 * _NS + r, pl.ds(h * 16, 16)] = (
                    outbuf[0, pl.ds(h * 16, 16)])
        pltpu.async_copy(outbuf, out.at[oidx], sem).wait()
        return carry

    lax.fori_loop(0, 63, roi_body, 0)


_roi_align_sc = functools.partial(
    pl.kernel,
    out_type=jax.ShapeDtypeStruct((_OUTROWS, 128), jnp.float32),
    mesh=plsc.VectorSubcoreMesh(core_axis_name="c", subcore_axis_name="s"),
    compiler_params=pltpu.CompilerParams(needs_layout_passes=False),
    scratch_types=[
        pltpu.VMEM((64 * 16,), jnp.float32),
        pltpu.VMEM((_ROWS_PAD,), jnp.int32),
        pltpu.VMEM((_OIDX_PAD,), jnp.int32),
        pltpu.VMEM((4 * 64,), jnp.float32),
        pltpu.VMEM((_ROWS_PAD, 2 * _C), jnp.float32),
        pltpu.VMEM((_OIDX_PAD, 128), jnp.float32),
        pltpu.SemaphoreType.DMA,
    ],
)(_roi_align_body)


def kernel(features, rois):
    B, C, H, W = features.shape
    n = rois.shape[0]
    ftab = _relayout_tc(features.reshape(B, C, H * W))
    roisp = jnp.zeros((2048, 16), jnp.float32).at[:n, :5].set(rois).reshape(-1)
    out = _roi_align_sc(ftab, roisp)
    out = out.reshape(_NS, _NROI // 8, 2, 8, 128)
    out = jnp.transpose(out, (1, 3, 2, 4, 0))
    return out.reshape(n, C, _AH, _AW)


# speedup vs baseline: 5.7129x; 1.0202x over previous
import functools

import jax
import jax.numpy as jnp
from jax import lax
from jax.experimental import pallas as pl
from jax.experimental.pallas import tpu as pltpu
from jax.experimental.pallas import tpu_sc as plsc

_SCALE = 0.0625
_AH = 7
_AW = 7
_NS = _AH * _AW
_B, _C, _H, _W = 2, 256, 64, 64
_NPIX = _B * _H * _W
_NROI = 2000
_ROWS_PAD = 104
_OIDX_PAD = 104
_OUTROWS = _NS * (_NROI // 8) * 2 * 8


def _relayout_body(fin, fout):
    t = fin[0].T
    tshift = jnp.concatenate([t[_W:], t[:_W]], axis=0)
    fout[...] = jnp.concatenate([t, tshift], axis=1)


_relayout_tc = pl.pallas_call(
    _relayout_body,
    grid=(_B,),
    in_specs=[pl.BlockSpec((1, _C, _H * _W), lambda b: (b, 0, 0))],
    out_specs=pl.BlockSpec((_H * _W, 2 * _C), lambda b: (b, 0)),
    out_shape=jax.ShapeDtypeStruct((_NPIX, 2 * _C), jnp.float32),
)


def _roi_align_body(ftab, roisp, out, roibuf, idx0, idx1, oidx, wb0, wb1,
                    gb0, gb1, outbuf, semg0, semg1, semo):
    cid = lax.axis_index("c")
    sid = lax.axis_index("s")
    wid = sid * 2 + cid
    start = wid * 62 + jnp.minimum(wid, 16)
    pltpu.sync_copy(roisp.at[pl.ds(start * 16, 64 * 16)], roibuf)

    iota = lax.iota(jnp.int32, 16)
    idx0[pl.ds(_ROWS_PAD - 16, 16)] = jnp.zeros((16,), jnp.int32)
    idx1[pl.ds(_ROWS_PAD - 16, 16)] = jnp.zeros((16,), jnp.int32)

    def _splat_load(ref, i):
        return plsc.load_gather(ref, [jnp.broadcast_to(i, (16,))])

    def prep(j, idxb, wbuf):
        n = jnp.minimum(start + j, _NROI - 1)
        local = (n - start) * 16
        bv = _splat_load(roibuf, local).astype(jnp.int32)
        x1 = _splat_load(roibuf, local + 1) * _SCALE
        y1 = _splat_load(roibuf, local + 2) * _SCALE
        x2 = _splat_load(roibuf, local + 3) * _SCALE
        y2 = _splat_load(roibuf, local + 4) * _SCALE
        binh = jnp.maximum(y2 - y1 + 1.0, 0.0) * (1.0 / (_AH - 1))
        binw = jnp.maximum(x2 - x1 + 1.0, 0.0) * (1.0 / (_AW - 1))
        for r in range(4):
            s = iota + 16 * r
            ph = (s // _AW).astype(jnp.float32)
            pw = (s % _AW).astype(jnp.float32)
            hs = y1 + ph * binh
            ws = x1 + pw * binw
            valid = (hs >= 0.0) & (hs < float(_H)) & (ws >= 0.0) & (ws < float(_W))
            hst = jnp.clip(hs.astype(jnp.int32), 0, _H - 2)
            wst = jnp.clip(ws.astype(jnp.int32), 0, _W - 2)
            hr = hs - hst.astype(jnp.float32)
            wr = ws - wst.astype(jnp.float32)
            vf = jnp.where(valid, 1.0, 0.0)
            omh = (1.0 - hr) * vf
            hrv = hr * vf
            wbuf[pl.ds(16 * r, 16)] = omh * (1.0 - wr)
            wbuf[pl.ds(64 + 16 * r, 16)] = omh * wr
            wbuf[pl.ds(128 + 16 * r, 16)] = hrv * (1.0 - wr)
            wbuf[pl.ds(192 + 16 * r, 16)] = hrv * wr
            p = bv * (_H * _W) + hst * _W + wst
            m = s < _NS
            plsc.store_scatter(idxb, [s], p, mask=m)
            plsc.store_scatter(idxb, [s + _NS], p + 1, mask=m)

    def fire(idxb, gbuf, sem):
        pltpu.async_copy(ftab.at[idxb], gbuf, sem)

    def drain(idxb, gbuf, sem):
        pltpu.make_async_copy(ftab.at[idxb], gbuf, sem).wait()

    def consume(j, wbuf, gbuf, wait_out=True):
        n = jnp.minimum(start + j, _NROI - 1)
        base_n = (n // 8) * 16 + (n % 8)
        for r in range(7):
            pos = iota + 16 * r
            val = (pos // 2) * (2 * _NROI) + (pos % 2) * 8 + base_n
            val = jnp.where(pos < 2 * _NS, val, base_n)
            plsc.store_scatter(oidx, [pos], val, mask=pos < _OIDX_PAD)

        @plsc.parallel_loop(0, _NS, 1, unroll=2)
        def s_body(s):
            w0 = _splat_load(wbuf, s)
            w1 = _splat_load(wbuf, s + 64)
            w2 = _splat_load(wbuf, s + 128)
            w3 = _splat_load(wbuf, s + 192)
            orow = s * 2
            for k in range(_C // 16):
                off = 16 * k
                ul = gbuf[s, pl.ds(off, 16)]
                ur = gbuf[s + _NS, pl.ds(off, 16)]
                ll = gbuf[s, pl.ds(_C + off, 16)]
                lr = gbuf[s + _NS, pl.ds(_C + off, 16)]
                acc = ul * w0 + ur * w1 + ll * w2 + lr * w3
                outbuf[orow + (k // 8), pl.ds((k % 8) * 16, 16)] = acc
        for r in range(6):
            for h in range(8):
                outbuf[2 * _NS + r, pl.ds(h * 16, 16)] = (
                    outbuf[0, pl.ds(h * 16, 16)])
        cp = pltpu.async_copy(outbuf, out.at[oidx], semo)
        if wait_out:
            cp.wait()

    def wait_semo():
        pltpu.make_async_copy(outbuf, out.at[oidx], semo).wait()

    prep(0, idx0, wb0)
    fire(idx0, gb0, semg0)
    drain(idx0, gb0, semg0)
    consume(0, wb0, gb0)

    def pair_body(i, carry):
        j = 2 * i + 1
        prep(j, idx1, wb1)
        fire(idx1, gb1, semg1)
        prep(j + 1, idx0, wb0)
        fire(idx0, gb0, semg0)
        drain(idx1, gb1, semg1)

        @pl.when(i > 0)
        def _():
            wait_semo()

        consume(j, wb1, gb1, wait_out=False)
        drain(idx0, gb0, semg0)
        wait_semo()
        consume(j + 1, wb0, gb0, wait_out=False)
        return carry

    lax.fori_loop(0, 31, pair_body, 0)
    wait_semo()


_roi_align_sc = functools.partial(
    pl.kernel,
    out_type=jax.ShapeDtypeStruct((_OUTROWS, 128), jnp.float32),
    mesh=plsc.VectorSubcoreMesh(core_axis_name="c", subcore_axis_name="s"),
    compiler_params=pltpu.CompilerParams(needs_layout_passes=False),
    scratch_types=[
        pltpu.VMEM((64 * 16,), jnp.float32),
        pltpu.VMEM((_ROWS_PAD,), jnp.int32),
        pltpu.VMEM((_ROWS_PAD,), jnp.int32),
        pltpu.VMEM((_OIDX_PAD,), jnp.int32),
        pltpu.VMEM((4 * 64,), jnp.float32),
        pltpu.VMEM((4 * 64,), jnp.float32),
        pltpu.VMEM((_ROWS_PAD, 2 * _C), jnp.float32),
        pltpu.VMEM((_ROWS_PAD, 2 * _C), jnp.float32),
        pltpu.VMEM((_OIDX_PAD, 128), jnp.float32),
        pltpu.SemaphoreType.DMA,
        pltpu.SemaphoreType.DMA,
        pltpu.SemaphoreType.DMA,
    ],
)(_roi_align_body)


def kernel(features, rois):
    B, C, H, W = features.shape
    n = rois.shape[0]
    ftab = _relayout_tc(features.reshape(B, C, H * W))
    roisp = jnp.zeros((2048, 16), jnp.float32).at[:n, :5].set(rois).reshape(-1)
    out = _roi_align_sc(ftab, roisp)
    out = out.reshape(_NS, _NROI // 8, 2, 8, 128)
    out = jnp.transpose(out, (1, 3, 2, 4, 0))
    return out.reshape(n, C, _AH, _AW)
